# baseline (speedup 1.0000x reference)
import jax
import jax.numpy as jnp
from jax import lax
from jax.experimental import pallas as pl
from jax.experimental.pallas import tpu as pltpu
from jax.experimental.pallas import tpu_sc as plsc

_N = 10000
_E = 320000
_DIN = 128
_HID = 16
_NCLS = 40
_R = 8
_ROWS = _R + 1
_K2 = 48

_NC, _NS = 2, 16
_NW = _NC * _NS
_EPAD = 327680
_EPW = _EPAD // _NW
_RPW = _EPW // 128
_CROWS = 8
_NCH = _RPW // _CROWS
_NPAD = 10240
_ZR = _NPAD // _NS

_BLK = 2000
_A_ROWS_C0 = 96
_WBLK = 160


def _sc_edge_kernel(KW):
  mesh = plsc.VectorSubcoreMesh(core_axis_name="c", subcore_axis_name="s",
                                num_cores=_NC, num_subcores=_NS)
  Q = KW // 16
  crows = 16 if KW == 16 else 8
  rows_c0 = _A_ROWS_C0
  nch0 = rows_c0 // crows
  nch1 = (2 * _RPW - rows_c0) // crows

  def body(table, src2, col2, w2, dst2, out,
           srcv, colv, dstv, wv, gidxa, dsta, rows, agg, sem, sem2):
    cid = lax.axis_index("c")
    sid = lax.axis_index("s")
    wid = sid * _NC + cid
    zero16 = jnp.zeros((16,), jnp.float32)

    def zbody(i, c):
      for q in range(Q):
        rows[i, pl.ds(q * 16, 16)] = zero16
      return c
    lax.fori_loop(0, _ZR, zbody, 0, unroll=8)
    pltpu.sync_copy(rows.at[pl.ds(0, _ZR)], agg.at[pl.ds(sid * _ZR, _ZR)])
    plsc.subcore_barrier()

    row0 = sid * 2 * _RPW + jnp.where(cid == 0, 0, rows_c0)
    nch = jnp.where(cid == 0, nch0, nch1)

    def chunk(ch, c):
      rowbase = row0 + ch * crows
      pltpu.sync_copy(src2.at[pl.ds(rowbase, crows)], srcv)
      pltpu.sync_copy(col2.at[pl.ds(rowbase, crows)], colv)
      pltpu.sync_copy(dst2.at[pl.ds(rowbase, crows)], dstv)
      pltpu.sync_copy(w2.at[pl.ds(rowbase, crows)], wv)
      for i in range(crows):
        for j in range(8):
          sl = pl.ds(j * 16, 16)
          gidxa[i, sl] = srcv[i, sl] * _ROWS + colv[i, sl]
          dsta[i, sl] = dstv[i, sl]
      gd = [pltpu.async_copy(table.at[gidxa.at[i]],
                             rows.at[pl.ds(i * 128, 128)], sem)
            for i in range(crows)]
      sd = []
      for i in range(crows):
        gd[i].wait()

        def mul(g, cc, i=i):
          wg = wv[i, pl.ds(g * 16, 16)]
          for j in range(16):
            e = i * 128 + g * 16 + j
            wb = jnp.full((16,), wg[j], jnp.float32)
            for q in range(Q):
              rows[e, pl.ds(q * 16, 16)] = rows[e, pl.ds(q * 16, 16)] * wb
          return cc
        lax.fori_loop(0, 8, mul, 0)
        sd.append(pltpu.async_copy(rows.at[pl.ds(i * 128, 128)],
                                   agg.at[dsta.at[i]], sem2, add=True))
      for s_ in sd:
        s_.wait()
      return c
    lax.fori_loop(0, nch, chunk, 0)

    plsc.subcore_barrier()
    pltpu.sync_copy(agg.at[pl.ds(sid * _ZR, _ZR)],
                    out.at[pl.ds(cid * _NPAD + sid * _ZR, _ZR)])

  return pl.kernel(
      body,
      out_type=jax.ShapeDtypeStruct((_NC * _NPAD, KW), jnp.float32),
      mesh=mesh,
      compiler_params=pltpu.CompilerParams(use_tc_tiling_on_sc=False),
      scratch_types=[
          pltpu.VMEM((crows, 128), jnp.int32),
          pltpu.VMEM((crows, 128), jnp.int32),
          pltpu.VMEM((crows, 128), jnp.int32),
          pltpu.VMEM((crows, 128), jnp.float32),
          pltpu.VMEM((crows, 128), jnp.int32),
          pltpu.VMEM((crows, 128), jnp.int32),
          pltpu.VMEM((crows * 128, KW), jnp.float32),
          pltpu.VMEM_SHARED((_NPAD, KW), jnp.float32),
          pltpu.SemaphoreType.DMA,
          pltpu.SemaphoreType.DMA,
      ],
  )


def _mm_a(x, wflat):
  def k(x_ref, w_ref, o_ref):
    o_ref[...] = jnp.dot(x_ref[...], w_ref[...],
                         preferred_element_type=jnp.float32)
  return pl.pallas_call(
      k,
      grid=(_N // _BLK,),
      in_specs=[pl.BlockSpec((_BLK, _DIN), lambda i: (i, 0)),
                pl.BlockSpec((_DIN, _ROWS * _HID), lambda i: (0, 0))],
      out_specs=pl.BlockSpec((_BLK, _ROWS * _HID), lambda i: (i, 0)),
      out_shape=jax.ShapeDtypeStruct((_N, _ROWS * _HID), jnp.float32),
  )(x, wflat)


def _mm_b(aggs, t1, b1row, wflat2):
  def k(a_ref, t_ref, b_ref, w_ref, o_ref):
    h = a_ref[0] + a_ref[1] + t_ref[:, _R * _HID:] + b_ref[...]
    h = jnp.maximum(h, 0.0)
    o_ref[...] = jnp.dot(h, w_ref[...], preferred_element_type=jnp.float32)
  return pl.pallas_call(
      k,
      grid=(_N // _BLK,),
      in_specs=[pl.BlockSpec((_NC, _BLK, _HID), lambda i: (0, i, 0)),
                pl.BlockSpec((_BLK, _ROWS * _HID), lambda i: (i, 0)),
                pl.BlockSpec((1, _HID), lambda i: (0, 0)),
                pl.BlockSpec((_HID, _ROWS * _K2), lambda i: (0, 0))],
      out_specs=pl.BlockSpec((_BLK, _ROWS * _K2), lambda i: (i, 0)),
      out_shape=jax.ShapeDtypeStruct((_N, _ROWS * _K2), jnp.float32),
  )(aggs, t1, b1row, wflat2)


def _final_c(aggs2, t2, b2row):
  def k(a_ref, t_ref, b_ref, lp_ref, o_ref):
    o = a_ref[0] + a_ref[1] + t_ref[:, _R * _K2:] + b_ref[...]
    mask = lax.broadcasted_iota(jnp.int32, (1, _K2), 1) < _NCLS
    om = jnp.where(mask, o, -jnp.inf)
    mx = jnp.max(om, axis=1, keepdims=True)
    ex = jnp.where(mask, jnp.exp(o - mx), 0.0)
    lse = jnp.log(jnp.sum(ex, axis=1, keepdims=True)) + mx
    o_ref[...] = o[:, :_NCLS]
    lp_ref[...] = (o - lse)[:, :_NCLS]
  return pl.pallas_call(
      k,
      grid=(_N // _BLK,),
      in_specs=[pl.BlockSpec((_NC, _BLK, _K2), lambda i: (0, i, 0)),
                pl.BlockSpec((_BLK, _ROWS * _K2), lambda i: (i, 0)),
                pl.BlockSpec((1, _K2), lambda i: (0, 0))],
      out_specs=[pl.BlockSpec((_BLK, _NCLS), lambda i: (i, 0)),
                 pl.BlockSpec((_BLK, _NCLS), lambda i: (i, 0))],
      out_shape=[jax.ShapeDtypeStruct((_N, _NCLS), jnp.float32),
                 jax.ShapeDtypeStruct((_N, _NCLS), jnp.float32)],
  )(aggs2, t2, b2row)


def kernel(x, edge_index, edge_weight, edge_color, W1, root1, b1,
           W2, root2, b2):
  pad = _EPAD - _E
  src2 = jnp.concatenate(
      [edge_index[0], jnp.zeros((pad,), jnp.int32)]).reshape(-1, 128)
  dst2 = jnp.concatenate(
      [edge_index[1], jnp.zeros((pad,), jnp.int32)]).reshape(-1, 128)
  col2 = jnp.concatenate(
      [edge_color, jnp.zeros((pad,), jnp.int32)]).reshape(-1, 128)
  w2e = jnp.concatenate(
      [edge_weight, jnp.zeros((pad,), jnp.float32)]).reshape(-1, 128)

  wflat1 = jnp.concatenate(
      [W1.transpose(1, 0, 2).reshape(_DIN, _R * _HID), root1], axis=1)
  w2p = jnp.pad(W2, ((0, 0), (0, 0), (0, _K2 - _NCLS)))
  r2p = jnp.pad(root2, ((0, 0), (0, _K2 - _NCLS)))
  wflat2 = jnp.concatenate(
      [w2p.transpose(1, 0, 2).reshape(_HID, _R * _K2), r2p], axis=1)
  b1row = b1.reshape(1, _HID)
  b2row = jnp.pad(b2, (0, _K2 - _NCLS)).reshape(1, _K2)

  t1 = _mm_a(x, wflat1)
  table1 = t1.reshape(_N * _ROWS, _HID)
  aggs1 = _sc_edge_kernel(_HID)(table1, src2, col2, w2e, dst2)
  aggs1 = aggs1.reshape(_NC, _NPAD, _HID)[:, :_N]

  t2 = _mm_b(aggs1, t1, b1row, wflat2)
  table2 = t2.reshape(_N * _ROWS, _K2)
  aggs2 = _sc_edge_kernel(_K2)(table2, src2, col2, w2e, dst2)
  aggs2 = aggs2.reshape(_NC, _NPAD, _K2)[:, :_N]

  logp, out = _final_c(aggs2, t2, b2row)
  return (logp, out)


# speedup vs baseline: 1.0082x; 1.0082x over previous
import jax
import jax.numpy as jnp
from jax import lax
from jax.experimental import pallas as pl
from jax.experimental.pallas import tpu as pltpu
from jax.experimental.pallas import tpu_sc as plsc

_N = 10000
_E = 320000
_DIN = 128
_HID = 16
_NCLS = 40
_R = 8
_ROWS = _R + 1
_K2 = 48

_NC, _NS = 2, 16
_NW = _NC * _NS
_EPAD = 327680
_EPW = _EPAD // _NW
_RPW = _EPW // 128
_CROWS = 8
_NCH = _RPW // _CROWS
_NPAD = 10240
_ZR = _NPAD // _NS

_BLK = 2000
_A_ROWS_C0 = 104
_WBLK = 160


def _sc_edge_kernel(KW):
  mesh = plsc.VectorSubcoreMesh(core_axis_name="c", subcore_axis_name="s",
                                num_cores=_NC, num_subcores=_NS)
  Q = KW // 16
  crows = 8
  rows_c0 = _A_ROWS_C0
  nch0 = rows_c0 // crows
  nch1 = (2 * _RPW - rows_c0) // crows

  def body(table, src2, col2, w2, dst2, out,
           srcv, colv, dstv, wv, gidxa, dsta, rows, agg, sem, sem2):
    cid = lax.axis_index("c")
    sid = lax.axis_index("s")
    wid = sid * _NC + cid
    zero16 = jnp.zeros((16,), jnp.float32)

    def zbody(i, c):
      for q in range(Q):
        rows[i, pl.ds(q * 16, 16)] = zero16
      return c
    lax.fori_loop(0, _ZR, zbody, 0, unroll=8)
    pltpu.sync_copy(rows.at[pl.ds(0, _ZR)], agg.at[pl.ds(sid * _ZR, _ZR)])
    plsc.subcore_barrier()

    row0 = sid * 2 * _RPW + jnp.where(cid == 0, 0, rows_c0)
    nch = jnp.where(cid == 0, nch0, nch1)

    def chunk(ch, c):
      rowbase = row0 + ch * crows
      pltpu.sync_copy(src2.at[pl.ds(rowbase, crows)], srcv)
      pltpu.sync_copy(col2.at[pl.ds(rowbase, crows)], colv)
      pltpu.sync_copy(dst2.at[pl.ds(rowbase, crows)], dstv)
      pltpu.sync_copy(w2.at[pl.ds(rowbase, crows)], wv)
      for i in range(crows):
        for j in range(8):
          sl = pl.ds(j * 16, 16)
          gidxa[i, sl] = srcv[i, sl] * _ROWS + colv[i, sl]
          dsta[i, sl] = dstv[i, sl]
      gd = [pltpu.async_copy(table.at[gidxa.at[i]],
                             rows.at[pl.ds(i * 128, 128)], sem)
            for i in range(crows)]
      sd = []
      for i in range(crows):
        gd[i].wait()

        def mul(g, cc, i=i):
          wg = wv[i, pl.ds(g * 16, 16)]
          for j in range(16):
            e = i * 128 + g * 16 + j
            wb = jnp.full((16,), wg[j], jnp.float32)
            for q in range(Q):
              rows[e, pl.ds(q * 16, 16)] = rows[e, pl.ds(q * 16, 16)] * wb
          return cc
        lax.fori_loop(0, 8, mul, 0)
        sd.append(pltpu.async_copy(rows.at[pl.ds(i * 128, 128)],
                                   agg.at[dsta.at[i]], sem2, add=True))
      for s_ in sd:
        s_.wait()
      return c
    lax.fori_loop(0, nch, chunk, 0)

    plsc.subcore_barrier()
    pltpu.sync_copy(agg.at[pl.ds(sid * _ZR, _ZR)],
                    out.at[pl.ds(cid * _NPAD + sid * _ZR, _ZR)])

  return pl.kernel(
      body,
      out_type=jax.ShapeDtypeStruct((_NC * _NPAD, KW), jnp.float32),
      mesh=mesh,
      compiler_params=pltpu.CompilerParams(use_tc_tiling_on_sc=False),
      scratch_types=[
          pltpu.VMEM((crows, 128), jnp.int32),
          pltpu.VMEM((crows, 128), jnp.int32),
          pltpu.VMEM((crows, 128), jnp.int32),
          pltpu.VMEM((crows, 128), jnp.float32),
          pltpu.VMEM((crows, 128), jnp.int32),
          pltpu.VMEM((crows, 128), jnp.int32),
          pltpu.VMEM((crows * 128, KW), jnp.float32),
          pltpu.VMEM_SHARED((_NPAD, KW), jnp.float32),
          pltpu.SemaphoreType.DMA,
          pltpu.SemaphoreType.DMA,
      ],
  )


def _mm_a(x, wflat):
  def k(x_ref, w_ref, o_ref):
    o_ref[...] = jnp.dot(x_ref[...], w_ref[...],
                         preferred_element_type=jnp.float32)
  return pl.pallas_call(
      k,
      grid=(_N // _BLK,),
      in_specs=[pl.BlockSpec((_BLK, _DIN), lambda i: (i, 0)),
                pl.BlockSpec((_DIN, _ROWS * _HID), lambda i: (0, 0))],
      out_specs=pl.BlockSpec((_BLK, _ROWS * _HID), lambda i: (i, 0)),
      out_shape=jax.ShapeDtypeStruct((_N, _ROWS * _HID), jnp.float32),
  )(x, wflat)


def _mm_b(aggs, t1, b1row, wflat2):
  def k(a_ref, t_ref, b_ref, w_ref, o_ref):
    h = a_ref[0] + a_ref[1] + t_ref[:, _R * _HID:] + b_ref[...]
    h = jnp.maximum(h, 0.0)
    o_ref[...] = jnp.dot(h, w_ref[...], preferred_element_type=jnp.float32)
  return pl.pallas_call(
      k,
      grid=(_N // _BLK,),
      in_specs=[pl.BlockSpec((_NC, _BLK, _HID), lambda i: (0, i, 0)),
                pl.BlockSpec((_BLK, _ROWS * _HID), lambda i: (i, 0)),
                pl.BlockSpec((1, _HID), lambda i: (0, 0)),
                pl.BlockSpec((_HID, _ROWS * _K2), lambda i: (0, 0))],
      out_specs=pl.BlockSpec((_BLK, _ROWS * _K2), lambda i: (i, 0)),
      out_shape=jax.ShapeDtypeStruct((_N, _ROWS * _K2), jnp.float32),
  )(aggs, t1, b1row, wflat2)


def _final_c(aggs2, t2, b2row):
  def k(a_ref, t_ref, b_ref, lp_ref, o_ref):
    o = a_ref[0] + a_ref[1] + t_ref[:, _R * _K2:] + b_ref[...]
    mask = lax.broadcasted_iota(jnp.int32, (1, _K2), 1) < _NCLS
    om = jnp.where(mask, o, -jnp.inf)
    mx = jnp.max(om, axis=1, keepdims=True)
    ex = jnp.where(mask, jnp.exp(o - mx), 0.0)
    lse = jnp.log(jnp.sum(ex, axis=1, keepdims=True)) + mx
    o_ref[...] = o[:, :_NCLS]
    lp_ref[...] = (o - lse)[:, :_NCLS]
  return pl.pallas_call(
      k,
      grid=(_N // _BLK,),
      in_specs=[pl.BlockSpec((_NC, _BLK, _K2), lambda i: (0, i, 0)),
                pl.BlockSpec((_BLK, _ROWS * _K2), lambda i: (i, 0)),
                pl.BlockSpec((1, _K2), lambda i: (0, 0))],
      out_specs=[pl.BlockSpec((_BLK, _NCLS), lambda i: (i, 0)),
                 pl.BlockSpec((_BLK, _NCLS), lambda i: (i, 0))],
      out_shape=[jax.ShapeDtypeStruct((_N, _NCLS), jnp.float32),
                 jax.ShapeDtypeStruct((_N, _NCLS), jnp.float32)],
  )(aggs2, t2, b2row)


def kernel(x, edge_index, edge_weight, edge_color, W1, root1, b1,
           W2, root2, b2):
  pad = _EPAD - _E
  src2 = jnp.concatenate(
      [edge_index[0], jnp.zeros((pad,), jnp.int32)]).reshape(-1, 128)
  dst2 = jnp.concatenate(
      [edge_index[1], jnp.zeros((pad,), jnp.int32)]).reshape(-1, 128)
  col2 = jnp.concatenate(
      [edge_color, jnp.zeros((pad,), jnp.int32)]).reshape(-1, 128)
  w2e = jnp.concatenate(
      [edge_weight, jnp.zeros((pad,), jnp.float32)]).reshape(-1, 128)

  wflat1 = jnp.concatenate(
      [W1.transpose(1, 0, 2).reshape(_DIN, _R * _HID), root1], axis=1)
  w2p = jnp.pad(W2, ((0, 0), (0, 0), (0, _K2 - _NCLS)))
  r2p = jnp.pad(root2, ((0, 0), (0, _K2 - _NCLS)))
  wflat2 = jnp.concatenate(
      [w2p.transpose(1, 0, 2).reshape(_HID, _R * _K2), r2p], axis=1)
  b1row = b1.reshape(1, _HID)
  b2row = jnp.pad(b2, (0, _K2 - _NCLS)).reshape(1, _K2)

  t1 = _mm_a(x, wflat1)
  table1 = t1.reshape(_N * _ROWS, _HID)
  aggs1 = _sc_edge_kernel(_HID)(table1, src2, col2, w2e, dst2)
  aggs1 = aggs1.reshape(_NC, _NPAD, _HID)[:, :_N]

  t2 = _mm_b(aggs1, t1, b1row, wflat2)
  table2 = t2.reshape(_N * _ROWS, _K2)
  aggs2 = _sc_edge_kernel(_K2)(table2, src2, col2, w2e, dst2)
  aggs2 = aggs2.reshape(_NC, _NPAD, _K2)[:, :_N]

  logp, out = _final_c(aggs2, t2, b2row)
  return (logp, out)


# speedup vs baseline: 1.0110x; 1.0028x over previous
import jax
import jax.numpy as jnp
from jax import lax
from jax.experimental import pallas as pl
from jax.experimental.pallas import tpu as pltpu
from jax.experimental.pallas import tpu_sc as plsc

_N = 10000
_E = 320000
_DIN = 128
_HID = 16
_NCLS = 40
_R = 8
_ROWS = _R + 1
_K2 = 48

_NC, _NS = 2, 16
_NW = _NC * _NS
_EPAD = 327680
_EPW = _EPAD // _NW
_RPW = _EPW // 128
_NPAD = 10240
_ZR = _NPAD // _NS

_BLK = 2000
_A_ROWS_C0 = 104


def _sc_edge_kernel(KW):
  mesh = plsc.VectorSubcoreMesh(core_axis_name="c", subcore_axis_name="s",
                                num_cores=_NC, num_subcores=_NS)
  Q = KW // 16
  crows = 8
  rows_c0 = _A_ROWS_C0
  nch0 = rows_c0 // crows
  nch1 = (2 * _RPW - rows_c0) // crows

  def body(table, src2, col2, w2, dst2, out,
           srcv, colv, dstv, wv, gidxa, dsta, rows, agg, sem, sem2):
    cid = lax.axis_index("c")
    sid = lax.axis_index("s")
    wid = sid * _NC + cid
    zero16 = jnp.zeros((16,), jnp.float32)

    def zbody(i, c):
      for q in range(Q):
        rows[i, pl.ds(q * 16, 16)] = zero16
      return c
    lax.fori_loop(0, _ZR, zbody, 0, unroll=8)
    pltpu.sync_copy(rows.at[pl.ds(0, _ZR)], agg.at[pl.ds(sid * _ZR, _ZR)])
    plsc.subcore_barrier()

    row0 = sid * 2 * _RPW + jnp.where(cid == 0, 0, rows_c0)
    nch = jnp.where(cid == 0, nch0, nch1)

    def chunk(ch, c):
      rowbase = row0 + ch * crows
      pltpu.sync_copy(src2.at[pl.ds(rowbase, crows)], srcv)
      pltpu.sync_copy(col2.at[pl.ds(rowbase, crows)], colv)
      pltpu.sync_copy(dst2.at[pl.ds(rowbase, crows)], dstv)
      pltpu.sync_copy(w2.at[pl.ds(rowbase, crows)], wv)
      for i in range(crows):
        for j in range(8):
          sl = pl.ds(j * 16, 16)
          gidxa[i, sl] = srcv[i, sl] * _ROWS + colv[i, sl]
          dsta[i, sl] = dstv[i, sl]
      gd = [pltpu.async_copy(table.at[gidxa.at[i]],
                             rows.at[pl.ds(i * 128, 128)], sem)
            for i in range(crows)]
      sd = []
      for i in range(crows):
        gd[i].wait()

        def mul(g, cc, i=i):
          wg = wv[i, pl.ds(g * 16, 16)]
          for j in range(16):
            e = i * 128 + g * 16 + j
            wb = jnp.full((16,), wg[j], jnp.float32)
            for q in range(Q):
              rows[e, pl.ds(q * 16, 16)] = rows[e, pl.ds(q * 16, 16)] * wb
          return cc
        lax.fori_loop(0, 8, mul, 0)
        sd.append(pltpu.async_copy(rows.at[pl.ds(i * 128, 128)],
                                   agg.at[dsta.at[i]], sem2, add=True))
      for s_ in sd:
        s_.wait()
      return c
    lax.fori_loop(0, nch, chunk, 0)

    plsc.subcore_barrier()
    pltpu.sync_copy(agg.at[pl.ds(sid * _ZR, _ZR)],
                    out.at[pl.ds(cid * _NPAD + sid * _ZR, _ZR)])

  return pl.kernel(
      body,
      out_type=jax.ShapeDtypeStruct((_NC * _NPAD, KW), jnp.float32),
      mesh=mesh,
      compiler_params=pltpu.CompilerParams(use_tc_tiling_on_sc=False),
      scratch_types=[
          pltpu.VMEM((crows, 128), jnp.int32),
          pltpu.VMEM((crows, 128), jnp.int32),
          pltpu.VMEM((crows, 128), jnp.int32),
          pltpu.VMEM((crows, 128), jnp.float32),
          pltpu.VMEM((crows, 128), jnp.int32),
          pltpu.VMEM((crows, 128), jnp.int32),
          pltpu.VMEM((crows * 128, KW), jnp.float32),
          pltpu.VMEM_SHARED((_NPAD, KW), jnp.float32),
          pltpu.SemaphoreType.DMA,
          pltpu.SemaphoreType.DMA,
      ],
  )


def _mm_a(x, wflat):
  def k(x_ref, w_ref, o_ref):
    o_ref[...] = jnp.dot(x_ref[...], w_ref[...],
                         preferred_element_type=jnp.float32)
  return pl.pallas_call(
      k,
      grid=(_N // _BLK,),
      in_specs=[pl.BlockSpec((_BLK, _DIN), lambda i: (i, 0)),
                pl.BlockSpec((_DIN, _ROWS * _HID), lambda i: (0, 0))],
      out_specs=pl.BlockSpec((_BLK, _ROWS * _HID), lambda i: (i, 0)),
      out_shape=jax.ShapeDtypeStruct((_N, _ROWS * _HID), jnp.float32),
  )(x, wflat)


def _mm_b(aggs, t1, b1row, wflat2):
  def k(a_ref, t_ref, b_ref, w_ref, o_ref):
    h = a_ref[0] + a_ref[1] + t_ref[:, _R * _HID:] + b_ref[...]
    h = jnp.maximum(h, 0.0)
    o_ref[...] = jnp.dot(h, w_ref[...], preferred_element_type=jnp.float32)
  return pl.pallas_call(
      k,
      grid=(_N // _BLK,),
      in_specs=[pl.BlockSpec((_NC, _BLK, _HID), lambda i: (0, i, 0)),
                pl.BlockSpec((_BLK, _ROWS * _HID), lambda i: (i, 0)),
                pl.BlockSpec((1, _HID), lambda i: (0, 0)),
                pl.BlockSpec((_HID, _ROWS * _K2), lambda i: (0, 0))],
      out_specs=pl.BlockSpec((_BLK, _ROWS * _K2), lambda i: (i, 0)),
      out_shape=jax.ShapeDtypeStruct((_N, _ROWS * _K2), jnp.float32),
  )(aggs, t1, b1row, wflat2)


def _final_c(aggs2, t2, b2row):
  def k(a_ref, t_ref, b_ref, lp_ref, o_ref):
    o = a_ref[0] + a_ref[1] + t_ref[:, _R * _K2:] + b_ref[...]
    mask = lax.broadcasted_iota(jnp.int32, (1, _K2), 1) < _NCLS
    om = jnp.where(mask, o, -jnp.inf)
    mx = jnp.max(om, axis=1, keepdims=True)
    ex = jnp.where(mask, jnp.exp(o - mx), 0.0)
    lse = jnp.log(jnp.sum(ex, axis=1, keepdims=True)) + mx
    o_ref[...] = o[:, :_NCLS]
    lp_ref[...] = (o - lse)[:, :_NCLS]
  return pl.pallas_call(
      k,
      grid=(_N // _BLK,),
      in_specs=[pl.BlockSpec((_NC, _BLK, _K2), lambda i: (0, i, 0)),
                pl.BlockSpec((_BLK, _ROWS * _K2), lambda i: (i, 0)),
                pl.BlockSpec((1, _K2), lambda i: (0, 0))],
      out_specs=[pl.BlockSpec((_BLK, _NCLS), lambda i: (i, 0)),
                 pl.BlockSpec((_BLK, _NCLS), lambda i: (i, 0))],
      out_shape=[jax.ShapeDtypeStruct((_N, _NCLS), jnp.float32),
                 jax.ShapeDtypeStruct((_N, _NCLS), jnp.float32)],
  )(aggs2, t2, b2row)


def kernel(x, edge_index, edge_weight, edge_color, W1, root1, b1,
           W2, root2, b2):
  pad = _EPAD - _E
  src2 = jnp.concatenate(
      [edge_index[0], jnp.zeros((pad,), jnp.int32)]).reshape(-1, 128)
  dst2 = jnp.concatenate(
      [edge_index[1], jnp.zeros((pad,), jnp.int32)]).reshape(-1, 128)
  col2 = jnp.concatenate(
      [edge_color, jnp.zeros((pad,), jnp.int32)]).reshape(-1, 128)
  w2e = jnp.concatenate(
      [edge_weight, jnp.zeros((pad,), jnp.float32)]).reshape(-1, 128)

  wflat1 = jnp.concatenate(
      [W1.transpose(1, 0, 2).reshape(_DIN, _R * _HID), root1], axis=1)
  w2p = jnp.pad(W2, ((0, 0), (0, 0), (0, _K2 - _NCLS)))
  r2p = jnp.pad(root2, ((0, 0), (0, _K2 - _NCLS)))
  wflat2 = jnp.concatenate(
      [w2p.transpose(1, 0, 2).reshape(_HID, _R * _K2), r2p], axis=1)
  b1row = b1.reshape(1, _HID)
  b2row = jnp.pad(b2, (0, _K2 - _NCLS)).reshape(1, _K2)

  t1 = _mm_a(x, wflat1)
  table1 = t1.reshape(_N * _ROWS, _HID)
  aggs1 = _sc_edge_kernel(_HID)(table1, src2, col2, w2e, dst2)
  aggs1 = aggs1.reshape(_NC, _NPAD, _HID)[:, :_N]

  t2 = _mm_b(aggs1, t1, b1row, wflat2)
  table2 = t2.reshape(_N * _ROWS, _K2)
  aggs2 = _sc_edge_kernel(_K2)(table2, src2, col2, w2e, dst2)
  aggs2 = aggs2.reshape(_NC, _NPAD, _K2)[:, :_N]

  logp, out = _final_c(aggs2, t2, b2row)
  return (logp, out)
